# baseline (speedup 1.0000x reference)
import jax
from jax.experimental import pallas as pl
from jax.experimental.pallas import tpu as pltpu


def _copy_all(h_in, r_in, t_in, m_in, h_out, r_out, t_out, m_out, sems):
    pairs = ((h_in, h_out), (r_in, r_out), (t_in, t_out), (m_in, m_out))
    copies = [
        pltpu.make_async_copy(src, dst, sems.at[i])
        for i, (src, dst) in enumerate(pairs)
    ]
    for c in copies:
        c.start()
    for c in copies:
        c.wait()


def kernel(head_w, relation_w, tail_w, r_mat):
    flats = tuple(x.reshape(-1) for x in (head_w, relation_w, tail_w, r_mat))
    out_shape = tuple(jax.ShapeDtypeStruct(x.shape, x.dtype) for x in flats)
    outs = pl.pallas_call(
        _copy_all,
        out_shape=out_shape,
        in_specs=[pl.BlockSpec(memory_space=pl.ANY)] * 4,
        out_specs=tuple(pl.BlockSpec(memory_space=pl.ANY) for _ in range(4)),
        scratch_shapes=[pltpu.SemaphoreType.DMA((4,))],
    )(*flats)
    return tuple(
        o.reshape(x.shape)
        for o, x in zip(outs, (head_w, relation_w, tail_w, r_mat))
    )


# speedup vs baseline: 4.6075x; 4.6075x over previous
import jax
from jax.experimental import pallas as pl
from jax.experimental.pallas import tpu as pltpu

_GRID = 25


def _copy_body(h_in, r_in, t_in, m_in, h_out, r_out, t_out, m_out):
    t_out[...] = t_in[...]
    h_out[...] = h_in[...]
    r_out[...] = r_in[...]
    m_out[...] = m_in[...]


def kernel(head_w, relation_w, tail_w, r_mat):
    orig = (head_w, relation_w, tail_w, r_mat)
    wides = tuple(x.reshape(-1, 128) for x in orig)
    hw, rw, tw, mw = wides

    def row_spec(arr):
        rows = arr.shape[0] // _GRID
        if rows >= 8:
            return pl.BlockSpec((rows, 128), lambda i: (i, 0))
        return pl.BlockSpec(arr.shape, lambda i: (0, 0))

    specs = [row_spec(a) for a in wides]
    out_shape = tuple(jax.ShapeDtypeStruct(x.shape, x.dtype) for x in wides)
    outs = pl.pallas_call(
        _copy_body,
        grid=(_GRID,),
        in_specs=specs,
        out_specs=tuple(specs),
        out_shape=out_shape,
        compiler_params=pltpu.CompilerParams(
            dimension_semantics=("arbitrary",),
        ),
    )(*wides)
    return tuple(o.reshape(x.shape) for o, x in zip(outs, orig))
